# Initial kernel scaffold; baseline (speedup 1.0000x reference)
import jax
import jax.numpy as jnp
from jax.experimental import pallas as pl


def kernel(x, w0, w1, w2, w3, b0, b1, b2, b3):
    raise NotImplementedError("write your pallas kernel here")


# speedup vs baseline: 1.1107x; 1.1107x over previous
import functools

import jax
import jax.numpy as jnp
from jax.experimental import pallas as pl
from jax.experimental.pallas import tpu as pltpu


def _conv_kernel(x_ref, w_ref, b_ref, o_ref, s_ref, *, TB, K, Cin, Wp, M, MQ):
    cout = o_ref.shape[1]
    bias = jnp.broadcast_to(b_ref[...], (cout, M)).astype(jnp.float32)
    for b in range(TB):
        for dy in range(K):
            s_ref[dy * Cin:(dy + 1) * Cin, :] = x_ref[b, :, dy * Wp:dy * Wp + MQ]
        acc = bias
        for dx in range(K):
            p = jnp.dot(w_ref[dx], s_ref[...],
                        preferred_element_type=jnp.float32)
            acc = acc + jax.lax.slice(p, (0, dx), (cout, dx + M))
        o_ref[b] = acc


def _largest_divisor_at_most(n, cap):
    for d in range(min(n, cap), 0, -1):
        if n % d == 0:
            return d
    return 1


def _inception_fused(x_nchw, weights, biases, batch_block=4):
    num_kernels = len(weights)
    n, cin, h, w = x_nchw.shape
    cout = weights[0].shape[0]
    P = num_kernels - 1
    K = 2 * P + 1
    Wp = w + 2 * P
    M = h * Wp
    MQ = M + K - 1
    cin_p = ((cin + 15) // 16) * 16
    L = (h + 2 * P + 1) * Wp

    w_avg = jnp.zeros((K, K, cin_p, cout), jnp.float32)
    for i, wi in enumerate(weights):
        k = 2 * i + 1
        off = P - i
        w_hwio = jnp.transpose(wi.astype(jnp.float32), (2, 3, 1, 0))
        w_avg = w_avg.at[off:off + k, off:off + k, :cin, :].add(w_hwio)
    w_avg = w_avg / num_kernels
    w_slab = jnp.transpose(w_avg, (1, 3, 0, 2)).reshape(K, cout, K * cin_p)
    w_slab = w_slab.astype(jnp.bfloat16)
    b_avg = (sum(b.astype(jnp.float32) for b in biases) / num_kernels)
    b_avg = b_avg.reshape(cout, 1)

    x_pad = jnp.pad(x_nchw.astype(jnp.float32),
                    ((0, 0), (0, cin_p - cin), (P, P + 1), (P, P)))
    x_flat = x_pad.reshape(n, cin_p, L).astype(jnp.bfloat16)

    TB = _largest_divisor_at_most(n, batch_block)
    kernel_fn = functools.partial(_conv_kernel, TB=TB, K=K, Cin=cin_p,
                                  Wp=Wp, M=M, MQ=MQ)

    out_wide = pl.pallas_call(
        kernel_fn,
        out_shape=jax.ShapeDtypeStruct((n, cout, M), jnp.float32),
        grid_spec=pltpu.PrefetchScalarGridSpec(
            num_scalar_prefetch=0,
            grid=(n // TB,),
            in_specs=[
                pl.BlockSpec((TB, cin_p, L), lambda g: (g, 0, 0)),
                pl.BlockSpec((K, cout, K * cin_p), lambda g: (0, 0, 0)),
                pl.BlockSpec((cout, 1), lambda g: (0, 0)),
            ],
            out_specs=pl.BlockSpec((TB, cout, M), lambda g: (g, 0, 0)),
            scratch_shapes=[pltpu.VMEM((K * cin_p, MQ), jnp.bfloat16)],
        ),
        compiler_params=pltpu.CompilerParams(
            dimension_semantics=("parallel",),
            vmem_limit_bytes=64 * 1024 * 1024),
    )(x_flat, w_slab, b_avg)

    return out_wide.reshape(n, cout, h, Wp)[:, :, :, :w]


def kernel(x, w0, w1, w2, w3, b0, b1, b2, b3):
    return _inception_fused(x, [w0, w1, w2, w3], [b0, b1, b2, b3])
